# baseline (speedup 1.0000x reference)
import functools

import jax
import jax.numpy as jnp
from jax import lax
from jax.experimental import pallas as pl
from jax.experimental.pallas import tpu as pltpu
from jax.experimental.pallas import tpu_sc as plsc

_D = 64
_NC = 2
_NS = 16
_NW = _NC * _NS
_IW = 128
_L = 16


def _iota16():
    return lax.iota(jnp.int32, _L)


def _ones_mask():
    return _iota16() >= 0


@functools.lru_cache(maxsize=None)
def _build_transpose(v: int):
    vpad = -(-v // _IW) * _IW
    n_cols = vpad // _IW
    per_w = -(-n_cols // _NW)
    mesh = plsc.VectorSubcoreMesh(core_axis_name="c", subcore_axis_name="s")

    @functools.partial(
        pl.kernel,
        mesh=mesh,
        out_type=jax.ShapeDtypeStruct((vpad, _IW), jnp.float32),
        scratch_types=[
            pltpu.VMEM((2, _D, _IW), jnp.float32),
            pltpu.VMEM((2, _IW, _IW), jnp.float32),
            [pltpu.SemaphoreType.DMA] * 2,
            [pltpu.SemaphoreType.DMA] * 2,
        ],
        compiler_params=pltpu.CompilerParams(use_tc_tiling_on_sc=True, needs_layout_passes=False),
    )
    def k(tt_hbm, out_hbm, tv, tr, sem_i, sem_o):
        wid = lax.axis_index("s") * _NC + lax.axis_index("c")

        def cid(g):
            return wid * per_w + g

        def in_copy(b, c):
            off = pl.multiple_of(c * _IW, _IW)
            return pltpu.make_async_copy(
                tt_hbm.at[:, pl.ds(off, _IW)], tv.at[b], sem_i[b])

        def out_copy(b, c):
            off = pl.multiple_of(c * _IW, _IW)
            return pltpu.make_async_copy(
                tr.at[b], out_hbm.at[pl.ds(off, _IW), :], sem_o[b])

        @pl.when(cid(0) < n_cols)
        def _():
            in_copy(0, cid(0)).start()

        @pl.loop(0, per_w + (per_w % 2), step=2)
        def _(i):
            for b in range(2):
                g = i + b
                c = cid(g)

                @pl.when((g < per_w) & (c < n_cols))
                def _():
                    in_copy(b, c).wait()

                    @pl.when((g + 1 < per_w) & (cid(g + 1) < n_cols))
                    def _():
                        in_copy((b + 1) % 2, cid(g + 1)).start()

                    @pl.when(g >= 2)
                    def _():
                        out_copy(b, cid(g - 2)).wait()

                    @plsc.parallel_loop(0, _IW, unroll=16)
                    def _(r):
                        for q in range(_D // _L):
                            vals = plsc.load_gather(
                                tv.at[b],
                                [q * _L + _iota16(),
                                 jnp.full((_L,), r, jnp.int32)],
                            )
                            plsc.store_compressed(
                                tr.at[b, r, pl.ds(q * _L, _L)], vals,
                                mask=_ones_mask())

                    out_copy(b, c).start()

        for b in range(2):
            g_last = per_w - 2 + b
            if g_last >= 0:
                @pl.when(cid(g_last) < n_cols)
                def _():
                    out_copy(g_last % 2, cid(g_last)).wait()

    return k


@functools.lru_cache(maxsize=None)
def _build_gather(h: int, bsz: int, vpad: int):
    nblk = h * bsz // _IW
    per_w = nblk // _NW
    nbb = bsz // _IW
    _NB = 3
    _A = 1
    mesh = plsc.VectorSubcoreMesh(core_axis_name="c", subcore_axis_name="s")

    @functools.partial(
        pl.kernel,
        mesh=mesh,
        out_type=jax.ShapeDtypeStruct((h, _D, bsz), jnp.float32),
        scratch_types=[
            pltpu.VMEM((per_w, _IW), jnp.int32),
            pltpu.VMEM((_NB, _IW, _IW), jnp.float32),
            pltpu.VMEM((_NB, _D, _IW), jnp.float32),
            [pltpu.SemaphoreType.DMA] * _NB,
            [pltpu.SemaphoreType.DMA] * _NB,
        ],
        compiler_params=pltpu.CompilerParams(use_tc_tiling_on_sc=True, needs_layout_passes=False),
    )
    def k(idx_hbm, t128_hbm, out_hbm, idx_v, rows_v, col_v,
          sem_g, sem_o):
        wid = lax.axis_index("s") * _NC + lax.axis_index("c")
        base = pl.multiple_of(wid * per_w, 8)

        pltpu.sync_copy(idx_hbm.at[pl.ds(base, per_w), :], idx_v)

        def gather(b, g):
            return pltpu.make_async_copy(
                t128_hbm.at[idx_v.at[g]], rows_v.at[b], sem_g[b])

        def out_copy(b, g):
            blk = base + g
            hh = blk // nbb
            b0 = pl.multiple_of((blk % nbb) * _IW, _IW)
            return pltpu.make_async_copy(
                col_v.at[b], out_hbm.at[hh, :, pl.ds(b0, _IW)], sem_o[b])

        for g in range(_A):
            gather(g, g).start()

        @pl.loop(0, per_w, step=_NB)
        def _(i):
            for b in range(_NB):
                g = i + b

                @pl.when(g < per_w)
                def _():
                    @pl.when(g + _A < per_w)
                    def _():
                        gather((b + _A) % _NB, g + _A).start()

                    gather(b, g).wait()

                    @pl.when(g >= _NB)
                    def _():
                        out_copy(b, g - _NB).wait()

                    @plsc.parallel_loop(0, _D, unroll=16)
                    def _(d):
                        for q in range(_IW // _L):
                            vals = plsc.load_gather(
                                rows_v.at[b],
                                [q * _L + _iota16(),
                                 jnp.full((_L,), d, jnp.int32)],
                            )
                            plsc.store_compressed(
                                col_v.at[b, d, pl.ds(q * _L, _L)], vals,
                                mask=_ones_mask())

                    out_copy(b, g).start()

        for b in range(_NB):
            out_copy(b, per_w - _NB + b).wait()

    return k


def kernel(token_ids, table):
    bsz, h = token_ids.shape
    v, d = table.shape
    assert d == _D and (h * bsz) % (_IW * _NW) == 0 and bsz % _IW == 0
    vpad = -(-v // _IW) * _IW
    idx_t = token_ids.T.astype(jnp.int32).reshape(h * bsz // _IW, _IW)
    t128 = _build_transpose(v)(table.T)
    out_t = _build_gather(h, bsz, vpad)(idx_t, t128)
    return out_t.transpose(2, 0, 1)


# speedup vs baseline: 1.4493x; 1.4493x over previous
import functools

import jax
import jax.numpy as jnp
from jax import lax
from jax.experimental import pallas as pl
from jax.experimental.pallas import tpu as pltpu
from jax.experimental.pallas import tpu_sc as plsc

_D = 64
_NC = 2
_NS = 16
_NW = _NC * _NS
_IW = 128
_C = 256
_SUB = _C // _IW
_NB = 5
_A = 3


@functools.lru_cache(maxsize=None)
def _build(n_flat: int):
    b_per_w = n_flat // _NW
    n_chunks = b_per_w // _C
    assert n_chunks % _NB == 0 and _A < _NB
    mesh = plsc.VectorSubcoreMesh(core_axis_name="c", subcore_axis_name="s")

    @functools.partial(
        pl.kernel,
        mesh=mesh,
        out_type=jax.ShapeDtypeStruct((n_flat, _D), jnp.float32),
        scratch_types=[
            pltpu.VMEM((_NB, _SUB, _IW), jnp.int32),
            pltpu.VMEM((_NB, _C, _D), jnp.float32),
            [pltpu.SemaphoreType.DMA] * _NB,
            [pltpu.SemaphoreType.DMA] * _NB,
            [pltpu.SemaphoreType.DMA] * _NB,
        ],
        compiler_params=pltpu.CompilerParams(use_tc_tiling_on_sc=False),
    )
    def k(idx_hbm, table_hbm, out_hbm, idx_v, rows_v, sem_i, sem_g, sem_o):
        wid = lax.axis_index("s") * _NC + lax.axis_index("c")
        base = wid * b_per_w
        base_r = base // _IW

        def idx_copy(b, g):
            row = pl.multiple_of(base_r + g * _SUB, _SUB)
            return pltpu.make_async_copy(
                idx_hbm.at[pl.ds(row, _SUB)], idx_v.at[b], sem_i[b])

        def out_copy(b, g):
            off = pl.multiple_of(base + g * _C, _C)
            return pltpu.make_async_copy(
                rows_v.at[b], out_hbm.at[pl.ds(off, _C)], sem_o[b])

        def fire_gathers(b):
            for j in range(_SUB):
                pltpu.async_copy(
                    table_hbm.at[idx_v.at[b, j]],
                    rows_v.at[b, pl.ds(j * _IW, _IW)],
                    sem_g[b],
                )

        def drain_gathers(b):
            for j in range(_SUB):
                pltpu.make_async_copy(
                    table_hbm.at[idx_v.at[b, j]],
                    rows_v.at[b, pl.ds(j * _IW, _IW)],
                    sem_g[b],
                ).wait()

        for b in range(_NB):
            idx_copy(b, b).start()
        for g in range(_A):
            idx_copy(g, g).wait()
            fire_gathers(g)

        @pl.loop(0, n_chunks, step=_NB)
        def _(i):
            for b in range(_NB):
                g = i + b
                ba = (b + _A) % _NB

                @pl.when(g + _A < n_chunks)
                def _():
                    idx_copy(ba, g + _A).wait()

                    @pl.when(g + _A >= _NB)
                    def _():
                        out_copy(ba, g + _A - _NB).wait()

                    fire_gathers(ba)

                drain_gathers(b)
                out_copy(b, g).start()

                @pl.when(g + _NB < n_chunks)
                def _():
                    idx_copy(b, g + _NB).start()

        for b in range(_NB):
            out_copy(b, n_chunks - _NB + b).wait()

    return k


def kernel(token_ids, table):
    b, h = token_ids.shape
    flat = token_ids.reshape(-1).astype(jnp.int32)
    n = b * h
    step = _NW * _C * _NB
    n_pad = -(-n // step) * step
    if n_pad != n:
        flat = jnp.concatenate([flat, jnp.zeros((n_pad - n,), jnp.int32)])
    idx2d = flat.reshape(-1, _IW)
    out = _build(n_pad)(idx2d, table)
    return out[:n].reshape(b, h, _D)
